# baseline (speedup 1.0000x reference)
import functools

import jax
import jax.numpy as jnp
from jax import lax
from jax.experimental import pallas as pl
from jax.experimental.pallas import tpu as pltpu
from jax.experimental.pallas import tpu_sc as plsc

_NUM_EMBEDDINGS = 100000
_DIM = 128
_B = 4096
_S = 50
_NUM_WORKERS = 32
_B_PER_W = _B // _NUM_WORKERS
_SG = 2
_NG = _S // _SG
_NBUF = 3


_mesh = plsc.VectorSubcoreMesh(core_axis_name="c", subcore_axis_name="s")


@functools.partial(
    pl.kernel,
    mesh=_mesh,
    out_type=jax.ShapeDtypeStruct((_S, _B, _DIM), jnp.float32),
    scratch_types=[
        pltpu.VMEM((_S, _B_PER_W), jnp.int32),
        [pltpu.VMEM((_SG, _B_PER_W, _DIM), jnp.float32) for _ in range(_NBUF)],
        [pltpu.SemaphoreType.DMA for _ in range(_NBUF)],
        [pltpu.SemaphoreType.DMA for _ in range(_NBUF)],
    ],
    compiler_params=pltpu.CompilerParams(use_tc_tiling_on_sc=True),
)
def _gather_kernel(table_hbm, idx_hbm, out_hbm, idx_v, bufs, gsems, osems):
    wid = lax.axis_index("s") * 2 + lax.axis_index("c")
    base = wid * _B_PER_W

    pltpu.sync_copy(idx_hbm.at[:, pl.ds(base, _B_PER_W)], idx_v)

    def gathers(g, b):
        return [
            pltpu.make_async_copy(
                table_hbm.at[idx_v.at[g * _SG + j]], bufs[b].at[j], gsems[b])
            for j in range(_SG)
        ]

    def out_copy(g, b):
        return pltpu.make_async_copy(
            bufs[b], out_hbm.at[pl.ds(g * _SG, _SG), pl.ds(base, _B_PER_W)],
            osems[b])

    def step(g, jb, wait_o, issue_g):
        for gd in gathers(g, jb):
            gd.wait()
        if wait_o:
            out_copy(g - 2, (jb + 1) % _NBUF).wait()
        if issue_g:
            for gd in gathers(g + 1, (jb + 1) % _NBUF):
                gd.start()
        out_copy(g, jb).start()

    for gd in gathers(0, 0):
        gd.start()
    step(0, 0, wait_o=False, issue_g=True)
    step(1, 1, wait_o=False, issue_g=True)

    def body(k, _):
        g0 = 2 + _NBUF * k
        for j in range(_NBUF):
            step(g0 + j, (2 + j) % _NBUF, wait_o=True, issue_g=True)
        return ()

    n_loop_groups = (_NG - 2 - 2) // _NBUF
    lax.fori_loop(0, n_loop_groups, body, ())

    for g in range(2 + n_loop_groups * _NBUF, _NG):
        step(g, g % _NBUF, wait_o=True, issue_g=(g + 1 < _NG))
    out_copy(_NG - 2, (_NG - 2) % _NBUF).wait()
    out_copy(_NG - 1, (_NG - 1) % _NBUF).wait()


def kernel(x, weight):
    out = _gather_kernel(weight, x.T.astype(jnp.int32))
    return out.transpose(1, 0, 2)


# speedup vs baseline: 1.0462x; 1.0462x over previous
import functools

import jax
import jax.numpy as jnp
from jax import lax
from jax.experimental import pallas as pl
from jax.experimental.pallas import tpu as pltpu
from jax.experimental.pallas import tpu_sc as plsc

_NUM_EMBEDDINGS = 100000
_DIM = 128
_B = 4096
_S = 50
_NUM_WORKERS = 32
_B_PER_W = _B // _NUM_WORKERS
_NBUF = 6
_GAHEAD = 4


_mesh = plsc.VectorSubcoreMesh(core_axis_name="c", subcore_axis_name="s")


@functools.partial(
    pl.kernel,
    mesh=_mesh,
    out_type=jax.ShapeDtypeStruct((_S, _B, _DIM), jnp.float32),
    scratch_types=[
        pltpu.VMEM((_S, _B_PER_W), jnp.int32),
        [pltpu.VMEM((_B_PER_W, _DIM), jnp.float32) for _ in range(_NBUF)],
        [pltpu.SemaphoreType.DMA for _ in range(_NBUF)],
        [pltpu.SemaphoreType.DMA for _ in range(_NBUF)],
    ],
    compiler_params=pltpu.CompilerParams(use_tc_tiling_on_sc=True),
)
def _gather_kernel(table_hbm, idx_hbm, out_hbm, idx_v, bufs, gsems, osems):
    wid = lax.axis_index("s") * 2 + lax.axis_index("c")
    base = wid * _B_PER_W

    pltpu.sync_copy(idx_hbm.at[:, pl.ds(base, _B_PER_W)], idx_v)

    def gather(c, b):
        return pltpu.make_async_copy(
            table_hbm.at[idx_v.at[c]], bufs[b], gsems[b])

    def out_copy(c, b):
        return pltpu.make_async_copy(
            bufs[b], out_hbm.at[c, pl.ds(base, _B_PER_W)], osems[b])

    def step(c, jb, wait_o, issue_g):
        gather(c, jb).wait()
        if wait_o:
            out_copy(c - 2, (jb + _GAHEAD) % _NBUF).wait()
        if issue_g:
            gather(c + _GAHEAD, (jb + _GAHEAD) % _NBUF).start()
        out_copy(c, jb).start()

    for c in range(_GAHEAD + 1):
        gather(c, c).start()
    step(0, 0, wait_o=False, issue_g=False)
    gather(_GAHEAD + 1, _GAHEAD + 1).start()
    step(1, 1, wait_o=False, issue_g=False)

    def body(k, _):
        c0 = 2 + _NBUF * k
        for j in range(_NBUF):
            step(c0 + j, (2 + j) % _NBUF, wait_o=True, issue_g=True)
        return ()

    n_loop = (_S - 2 - _GAHEAD - 2) // _NBUF
    lax.fori_loop(0, n_loop, body, ())

    for c in range(2 + n_loop * _NBUF, _S):
        step(c, c % _NBUF, wait_o=True, issue_g=(c + _GAHEAD < _S))
    out_copy(_S - 2, (_S - 2) % _NBUF).wait()
    out_copy(_S - 1, (_S - 1) % _NBUF).wait()


def kernel(x, weight):
    out = _gather_kernel(weight, x.T.astype(jnp.int32))
    return out.transpose(1, 0, 2)


# speedup vs baseline: 1.0509x; 1.0045x over previous
import functools

import jax
import jax.numpy as jnp
from jax import lax
from jax.experimental import pallas as pl
from jax.experimental.pallas import tpu as pltpu
from jax.experimental.pallas import tpu_sc as plsc

_NUM_EMBEDDINGS = 100000
_DIM = 128
_B = 4096
_S = 50
_NUM_WORKERS = 32
_B_PER_W = _B // _NUM_WORKERS
_NBUF = 6
_GAHEAD = 4


_mesh = plsc.VectorSubcoreMesh(core_axis_name="c", subcore_axis_name="s")


@functools.partial(
    pl.kernel,
    mesh=_mesh,
    out_type=jax.ShapeDtypeStruct((_S, _B, _DIM), jnp.float32),
    scratch_types=[
        pltpu.VMEM((_S, _B_PER_W), jnp.int32),
        [pltpu.VMEM((_B_PER_W, _DIM), jnp.float32) for _ in range(_NBUF)],
        [pltpu.SemaphoreType.DMA for _ in range(_NBUF)],
        [pltpu.SemaphoreType.DMA for _ in range(_NBUF)],
    ],
    compiler_params=pltpu.CompilerParams(
        use_tc_tiling_on_sc=True,
        skip_device_barrier=True,
        disable_bounds_checks=True,
        disable_semaphore_checks=True,
    ),
)
def _gather_kernel(table_hbm, idx_hbm, out_hbm, idx_v, bufs, gsems, osems):
    wid = lax.axis_index("s") * 2 + lax.axis_index("c")
    base = wid * _B_PER_W

    pltpu.sync_copy(idx_hbm.at[:, pl.ds(base, _B_PER_W)], idx_v)

    def gather(c, b):
        return pltpu.make_async_copy(
            table_hbm.at[idx_v.at[c]], bufs[b], gsems[b])

    def out_copy(c, b):
        return pltpu.make_async_copy(
            bufs[b], out_hbm.at[c, pl.ds(base, _B_PER_W)], osems[b])

    def step(c, jb, wait_o, issue_g):
        gather(c, jb).wait()
        if wait_o:
            out_copy(c - 2, (jb + _GAHEAD) % _NBUF).wait()
        if issue_g:
            gather(c + _GAHEAD, (jb + _GAHEAD) % _NBUF).start()
        out_copy(c, jb).start()

    for c in range(_GAHEAD + 1):
        gather(c, c).start()
    step(0, 0, wait_o=False, issue_g=False)
    gather(_GAHEAD + 1, _GAHEAD + 1).start()
    step(1, 1, wait_o=False, issue_g=False)

    def body(k, _):
        c0 = 2 + _NBUF * k
        for j in range(_NBUF):
            step(c0 + j, (2 + j) % _NBUF, wait_o=True, issue_g=True)
        return ()

    n_loop = (_S - 2 - _GAHEAD - 2) // _NBUF
    lax.fori_loop(0, n_loop, body, ())

    for c in range(2 + n_loop * _NBUF, _S):
        step(c, c % _NBUF, wait_o=True, issue_g=(c + _GAHEAD < _S))
    out_copy(_S - 2, (_S - 2) % _NBUF).wait()
    out_copy(_S - 1, (_S - 1) % _NBUF).wait()


def kernel(x, weight):
    out = _gather_kernel(weight, x.T.astype(jnp.int32))
    return out.transpose(1, 0, 2)
